# Initial kernel scaffold; baseline (speedup 1.0000x reference)
import functools

import jax
import jax.numpy as jnp
from jax import lax
from jax.experimental import pallas as pl
from jax.experimental.pallas import tpu as pltpu
from jax.experimental.pallas import tpu_sc as plsc

N = 10000
D = 128
E = 320000

NC = 2
NS = 16
L = 16

NPAD = 10240
B = 128
NB = 158
EPT = NB * B
EPAD = EPT * NS
EPW = EPAD // (NC * NS)
RPT = NPAD // NS

_mesh = plsc.VectorSubcoreMesh(
    core_axis_name="c", subcore_axis_name="s", num_cores=NC, num_subcores=NS)


@functools.partial(
    pl.kernel,
    out_type=jax.ShapeDtypeStruct((NC * NS, 2, NPAD), jnp.float32),
    mesh=_mesh,
    scratch_types=[
        pltpu.VMEM((EPW,), jnp.int32),
        pltpu.VMEM((EPW,), jnp.int32),
        pltpu.VMEM((NPAD,), jnp.float32),
        pltpu.VMEM((NPAD,), jnp.float32),
    ],
)
def _deg_kernel(src_hbm, dst_hbm, out_hbm, src_v, dst_v, dout_v, din_v):
    c = lax.axis_index("c")
    s = lax.axis_index("s")
    wid = s * NC + c
    pltpu.sync_copy(src_hbm.at[wid], src_v)
    pltpu.sync_copy(dst_hbm.at[wid], dst_v)

    zero = jnp.zeros((L,), jnp.float32)

    @pl.loop(0, NPAD // L)
    def _zero(r):
        dout_v[pl.ds(r * L, L)] = zero
        din_v[pl.ds(r * L, L)] = zero

    ones = jnp.ones((L,), jnp.float32)

    @pl.loop(0, EPW // L)
    def _hist(i):
        si = src_v[pl.ds(i * L, L)]
        di = dst_v[pl.ds(i * L, L)]
        plsc.addupdate_scatter(dout_v, [si], ones)
        plsc.addupdate_scatter(din_v, [di], ones)

    pltpu.sync_copy(dout_v, out_hbm.at[wid, 0])
    pltpu.sync_copy(din_v, out_hbm.at[wid, 1])


_BR2 = 2048


def _scale_body(deg_ref, feat_ref, hx_ref, u_ref, nd_ref):
    deg = jnp.sum(deg_ref[...], axis=0)
    od = deg[0]
    ig = deg[1]
    ns = lax.rsqrt(jnp.where(od > 0, od, 1.0))
    nd = lax.rsqrt(jnp.where(ig > 0, ig, 1.0))
    u_ref[0] = feat_ref[...] * ns[:, None]
    u_ref[1] = hx_ref[...] * ns[:, None]
    nd_ref[0, :] = nd


def _scale(deg_parts, feat_pad, hx_pad):
    return pl.pallas_call(
        _scale_body,
        grid=(NPAD // _BR2,),
        in_specs=[
            pl.BlockSpec((NC * NS, 2, _BR2), lambda g: (0, 0, g)),
            pl.BlockSpec((_BR2, D), lambda g: (g, 0)),
            pl.BlockSpec((_BR2, D), lambda g: (g, 0)),
        ],
        out_specs=[
            pl.BlockSpec((2, _BR2, D), lambda g: (0, g, 0)),
            pl.BlockSpec((1, _BR2), lambda g: (0, g)),
        ],
        out_shape=[
            jax.ShapeDtypeStruct((2, NPAD, D), jnp.float32),
            jax.ShapeDtypeStruct((1, NPAD), jnp.float32),
        ],
    )(deg_parts, feat_pad, hx_pad)


@functools.partial(
    pl.kernel,
    out_type=jax.ShapeDtypeStruct((NC, NPAD, D), jnp.float32),
    mesh=_mesh,
    scratch_types=[
        pltpu.VMEM_SHARED((NPAD, D), jnp.float32),
        pltpu.VMEM((NB, B), jnp.int32),
        pltpu.VMEM((NB, B), jnp.int32),
        pltpu.VMEM((B, D), jnp.float32),
        pltpu.SemaphoreType.DMA,
    ],
)
def _agg_kernel(u_hbm, src_hbm, dst_hbm, zeros_hbm, out_hbm,
                acc, src_v, dst_v, rows, sem):
    c = lax.axis_index("c")
    s = lax.axis_index("s")
    table = u_hbm.at[c]
    pltpu.sync_copy(src_hbm.at[s], src_v)
    pltpu.sync_copy(dst_hbm.at[s], dst_v)
    pltpu.sync_copy(zeros_hbm.at[pl.ds(s * RPT, RPT)],
                    acc.at[pl.ds(s * RPT, RPT)])
    plsc.subcore_barrier()

    @pl.loop(0, NB)
    def _edges(j):
        pltpu.async_copy(table.at[src_v.at[j]], rows, sem).wait()
        pltpu.sync_copy(rows, acc.at[dst_v.at[j]], add=True)

    plsc.subcore_barrier()
    pltpu.sync_copy(acc.at[pl.ds(s * RPT, RPT)],
                    out_hbm.at[c, pl.ds(s * RPT, RPT)])


_BR4 = 2000


def _gru_body(agg_ref, nd_ref, hx_ref, wi_ref, bi_ref, wh_ref, bh_ref, out_ref):
    nd = nd_ref[0, :][:, None]
    af = agg_ref[0] * nd
    ah = agg_ref[1] * nd
    i = jnp.dot(af, wi_ref[...], preferred_element_type=jnp.float32) + bi_ref[...]
    h = jnp.dot(ah, wh_ref[...], preferred_element_type=jnp.float32) + bh_ref[...]
    i_r, i_z, i_n = jnp.split(i, 3, axis=-1)
    h_r, h_z, h_n = jnp.split(h, 3, axis=-1)
    r = jax.nn.sigmoid(i_r + h_r)
    z = jax.nn.sigmoid(i_z + h_z)
    n = jnp.tanh(i_n + r * h_n)
    out_ref[...] = (1.0 - z) * n + z * hx_ref[...]


def _gru(agg, norm_dst, hx, W_i, b_i, W_h, b_h):
    return pl.pallas_call(
        _gru_body,
        grid=(N // _BR4,),
        in_specs=[
            pl.BlockSpec((NC, _BR4, D), lambda g: (0, g, 0)),
            pl.BlockSpec((1, _BR4), lambda g: (0, g)),
            pl.BlockSpec((_BR4, D), lambda g: (g, 0)),
            pl.BlockSpec((D, 3 * D), lambda g: (0, 0)),
            pl.BlockSpec((1, 3 * D), lambda g: (0, 0)),
            pl.BlockSpec((D, 3 * D), lambda g: (0, 0)),
            pl.BlockSpec((1, 3 * D), lambda g: (0, 0)),
        ],
        out_specs=pl.BlockSpec((_BR4, D), lambda g: (g, 0)),
        out_shape=jax.ShapeDtypeStruct((N, D), jnp.float32),
    )(agg, norm_dst, hx, W_i, b_i, W_h, b_h)


def kernel(feat, hx, edge_index, W_i, b_i, W_h, b_h):
    ei = edge_index.astype(jnp.int32)
    pad = EPAD - E
    src = jnp.concatenate([ei[0], jnp.full((pad,), N, jnp.int32)])
    dst = jnp.concatenate([ei[1], jnp.full((pad,), N, jnp.int32)])
    src2 = src.reshape(NC * NS, EPW)
    dst2 = dst.reshape(NC * NS, EPW)
    src3 = src.reshape(NS, NB, B)
    dst3 = dst.reshape(NS, NB, B)

    feat_pad = jnp.pad(feat, ((0, NPAD - N), (0, 0)))
    hx_pad = jnp.pad(hx, ((0, NPAD - N), (0, 0)))
    zeros = jnp.zeros((NPAD, D), jnp.float32)

    deg_parts = _deg_kernel(src2, dst2)
    u, norm_dst = _scale(deg_parts, feat_pad, hx_pad)
    agg = _agg_kernel(u, src3, dst3, zeros)
    return _gru(agg, norm_dst, hx, W_i, b_i, W_h, b_h)


# speedup vs baseline: 8.8996x; 8.8996x over previous
import functools

import jax
import jax.numpy as jnp
from jax import lax
from jax._src import config as _config
from jax.experimental import pallas as pl
from jax.experimental.pallas import tpu as pltpu
from jax.experimental.pallas import tpu_sc as plsc

N = 10000
D = 128
E = 320000

NC = 2
NS = 16
L = 16

NPAD = 10240
B = 128
NB = 160
CH = 16
NCH = NB // CH
EPT = NB * B
EPAD = EPT * NS
EPW = EPAD // (NC * NS)
RPT = NPAD // NS


def _deg_body(src_hbm, dst_hbm, out_hbm, src_v, dst_v, dout_v, din_v):
    c = lax.axis_index("c")
    s = lax.axis_index("s")
    wid = s * jnp.int32(NC) + c
    pltpu.sync_copy(src_hbm.at[wid], src_v)
    pltpu.sync_copy(dst_hbm.at[wid], dst_v)

    zero = jnp.zeros((L,), jnp.float32)

    @pl.loop(0, NPAD // L)
    def _zero(r):
        off = r * jnp.int32(L)
        dout_v[pl.ds(off, L)] = zero
        din_v[pl.ds(off, L)] = zero

    ones = jnp.ones((L,), jnp.float32)

    @pl.loop(0, EPW // L)
    def _hist(i):
        off = i * jnp.int32(L)
        si = src_v[pl.ds(off, L)]
        di = dst_v[pl.ds(off, L)]
        plsc.addupdate_scatter(dout_v, [si], ones)
        plsc.addupdate_scatter(din_v, [di], ones)

    pltpu.sync_copy(dout_v, out_hbm.at[wid, 0])
    pltpu.sync_copy(din_v, out_hbm.at[wid, 1])


_BR2 = 2048


def _scale_body(deg_ref, feat_ref, hx_ref, u_ref, nd_ref):
    deg = jnp.sum(deg_ref[...], axis=0)
    od = deg[0]
    ig = deg[1]
    ns = lax.rsqrt(jnp.where(od > 0, od, 1.0))
    nd = lax.rsqrt(jnp.where(ig > 0, ig, 1.0))
    u_ref[0] = feat_ref[...] * ns[:, None]
    u_ref[1] = hx_ref[...] * ns[:, None]
    nd_ref[:, 0] = nd


def _scale(deg_parts, feat_pad, hx_pad):
    return pl.pallas_call(
        _scale_body,
        grid=(NPAD // _BR2,),
        in_specs=[
            pl.BlockSpec((NC * NS, 2, _BR2), lambda g: (0, 0, g)),
            pl.BlockSpec((_BR2, D), lambda g: (g, 0)),
            pl.BlockSpec((_BR2, D), lambda g: (g, 0)),
        ],
        out_specs=[
            pl.BlockSpec((2, _BR2, D), lambda g: (0, g, 0)),
            pl.BlockSpec((_BR2, 1), lambda g: (g, 0)),
        ],
        out_shape=[
            jax.ShapeDtypeStruct((2, NPAD, D), jnp.float32),
            jax.ShapeDtypeStruct((NPAD, 1), jnp.float32),
        ],
    )(deg_parts, feat_pad, hx_pad)


def _agg_body(u_hbm, src_hbm, dst_hbm, zeros_hbm, out_hbm,
              acc, src_v, dst_v, rows, sem):
    c = lax.axis_index("c")
    s = lax.axis_index("s")
    row0 = s * jnp.int32(RPT)
    table = u_hbm.at[c]
    pltpu.sync_copy(zeros_hbm.at[pl.ds(row0, RPT)],
                    acc.at[pl.ds(row0, RPT)])
    plsc.subcore_barrier()

    @pl.loop(0, NCH)
    def _chunk(ch):
        b0 = ch * jnp.int32(CH)
        pltpu.sync_copy(src_hbm.at[s, pl.ds(b0, CH)], src_v)
        pltpu.sync_copy(dst_hbm.at[s, pl.ds(b0, CH)], dst_v)
        for j in range(CH):
            pltpu.async_copy(table.at[src_v.at[j]], rows, sem).wait()
            pltpu.sync_copy(rows, acc.at[dst_v.at[j]], add=True)

    plsc.subcore_barrier()
    pltpu.sync_copy(acc.at[pl.ds(row0, RPT)],
                    out_hbm.at[c, pl.ds(row0, RPT)])


_BR4 = 2000


def _gru_body(agg_ref, nd_ref, hx_ref, wi_ref, bi_ref, wh_ref, bh_ref, out_ref):
    nd = nd_ref[...]
    af = agg_ref[0] * nd
    ah = agg_ref[1] * nd
    i = jnp.dot(af, wi_ref[...], preferred_element_type=jnp.float32) + bi_ref[...]
    h = jnp.dot(ah, wh_ref[...], preferred_element_type=jnp.float32) + bh_ref[...]
    i_r, i_z, i_n = jnp.split(i, 3, axis=-1)
    h_r, h_z, h_n = jnp.split(h, 3, axis=-1)
    r = jax.nn.sigmoid(i_r + h_r)
    z = jax.nn.sigmoid(i_z + h_z)
    n = jnp.tanh(i_n + r * h_n)
    out_ref[...] = (1.0 - z) * n + z * hx_ref[...]


def _gru(agg, norm_dst, hx, W_i, b_i, W_h, b_h):
    return pl.pallas_call(
        _gru_body,
        grid=(N // _BR4,),
        in_specs=[
            pl.BlockSpec((NC, _BR4, D), lambda g: (0, g, 0)),
            pl.BlockSpec((_BR4, 1), lambda g: (g, 0)),
            pl.BlockSpec((_BR4, D), lambda g: (g, 0)),
            pl.BlockSpec((D, 3 * D), lambda g: (0, 0)),
            pl.BlockSpec((1, 3 * D), lambda g: (0, 0)),
            pl.BlockSpec((D, 3 * D), lambda g: (0, 0)),
            pl.BlockSpec((1, 3 * D), lambda g: (0, 0)),
        ],
        out_specs=pl.BlockSpec((_BR4, D), lambda g: (g, 0)),
        out_shape=jax.ShapeDtypeStruct((N, D), jnp.float32),
    )(agg, norm_dst, hx, W_i, b_i, W_h, b_h)


@functools.cache
def _sc_kernels():
    mesh = plsc.VectorSubcoreMesh(
        core_axis_name="c", subcore_axis_name="s",
        num_cores=NC, num_subcores=NS)
    params = pltpu.CompilerParams(needs_layout_passes=False)
    deg_kernel = pl.kernel(
        _deg_body,
        out_type=jax.ShapeDtypeStruct((NC * NS, 2, NPAD), jnp.float32),
        mesh=mesh,
        compiler_params=params,
        scratch_types=[
            pltpu.VMEM((EPW,), jnp.int32),
            pltpu.VMEM((EPW,), jnp.int32),
            pltpu.VMEM((NPAD,), jnp.float32),
            pltpu.VMEM((NPAD,), jnp.float32),
        ],
    )
    agg_kernel = pl.kernel(
        _agg_body,
        out_type=jax.ShapeDtypeStruct((NC, NPAD, D), jnp.float32),
        mesh=mesh,
        compiler_params=params,
        scratch_types=[
            pltpu.VMEM_SHARED((NPAD, D), jnp.float32),
            pltpu.VMEM((CH, B), jnp.int32),
            pltpu.VMEM((CH, B), jnp.int32),
            pltpu.VMEM((B, D), jnp.float32),
            pltpu.SemaphoreType.DMA,
        ],
    )
    return deg_kernel, agg_kernel


def kernel(feat, hx, edge_index, W_i, b_i, W_h, b_h):
    with _config.enable_x64(False):
        out = _kernel_impl(feat, hx, edge_index, W_i, b_i, W_h, b_h)
    out_dtype = jnp.result_type(jnp.promote_types(W_i.dtype, feat.dtype))
    return out.astype(out_dtype)


def _kernel_impl(feat, hx, edge_index, W_i, b_i, W_h, b_h):
    feat = feat.astype(jnp.float32)
    hx = hx.astype(jnp.float32)
    W_i = W_i.astype(jnp.float32)
    b_i = b_i.astype(jnp.float32)
    W_h = W_h.astype(jnp.float32)
    b_h = b_h.astype(jnp.float32)
    ei = edge_index.astype(jnp.int32)
    pad = EPAD - E
    src = jnp.concatenate([ei[0], jnp.full((pad,), N, jnp.int32)])
    dst = jnp.concatenate([ei[1], jnp.full((pad,), N, jnp.int32)])
    src2 = src.reshape(NC * NS, EPW)
    dst2 = dst.reshape(NC * NS, EPW)
    src3 = src.reshape(NS, NB, B)
    dst3 = dst.reshape(NS, NB, B)

    feat_pad = jnp.pad(feat, ((0, NPAD - N), (0, 0)))
    hx_pad = jnp.pad(hx, ((0, NPAD - N), (0, 0)))
    zeros = jnp.zeros((NPAD, D), jnp.float32)

    deg_kernel, agg_kernel = _sc_kernels()
    deg_parts = deg_kernel(src2, dst2)
    u, norm_dst = _scale(deg_parts, feat_pad, hx_pad)
    agg = agg_kernel(u, src3, dst3, zeros)
    return _gru(agg, norm_dst, hx, W_i, b_i.reshape(1, 3 * D),
                W_h, b_h.reshape(1, 3 * D))


# speedup vs baseline: 10.3976x; 1.1683x over previous
import functools

import jax
import jax.numpy as jnp
from jax import lax
from jax._src import config as _config
from jax.experimental import pallas as pl
from jax.experimental.pallas import tpu as pltpu
from jax.experimental.pallas import tpu_sc as plsc

N = 10000
D = 128
E = 320000

NC = 2
NS = 16
L = 16

NPAD = 10240
B = 128
NB = 160
CH = 16
NCH = NB // CH
EPT = NB * B
EPAD = EPT * NS
EPW = EPAD // (NC * NS)
RPT = NPAD // NS


def _deg_body(src_hbm, dst_hbm, out_hbm, src_v, dst_v, dout_v, din_v):
    c = lax.axis_index("c")
    s = lax.axis_index("s")
    wid = s * jnp.int32(NC) + c
    pltpu.sync_copy(src_hbm.at[wid], src_v)
    pltpu.sync_copy(dst_hbm.at[wid], dst_v)

    zero = jnp.zeros((L,), jnp.float32)

    @pl.loop(0, NPAD // L)
    def _zero(r):
        off = r * jnp.int32(L)
        dout_v[pl.ds(off, L)] = zero
        din_v[pl.ds(off, L)] = zero

    ones = jnp.ones((L,), jnp.float32)

    @pl.loop(0, EPW // L)
    def _hist(i):
        off = i * jnp.int32(L)
        si = src_v[pl.ds(off, L)]
        di = dst_v[pl.ds(off, L)]
        plsc.addupdate_scatter(dout_v, [si], ones)
        plsc.addupdate_scatter(din_v, [di], ones)

    pltpu.sync_copy(dout_v, out_hbm.at[wid, 0])
    pltpu.sync_copy(din_v, out_hbm.at[wid, 1])


_BR2 = 2048


def _scale_body(deg_ref, feat_ref, hx_ref, u_ref, nd_ref):
    deg = jnp.sum(deg_ref[...], axis=0)
    od = deg[0]
    ig = deg[1]
    ns = lax.rsqrt(jnp.where(od > 0, od, 1.0))
    nd = lax.rsqrt(jnp.where(ig > 0, ig, 1.0))
    u_ref[0] = feat_ref[...] * ns[:, None]
    u_ref[1] = hx_ref[...] * ns[:, None]
    nd_ref[:, 0] = nd


def _scale(deg_parts, feat_pad, hx_pad):
    return pl.pallas_call(
        _scale_body,
        grid=(NPAD // _BR2,),
        in_specs=[
            pl.BlockSpec((NC * NS, 2, _BR2), lambda g: (0, 0, g)),
            pl.BlockSpec((_BR2, D), lambda g: (g, 0)),
            pl.BlockSpec((_BR2, D), lambda g: (g, 0)),
        ],
        out_specs=[
            pl.BlockSpec((2, _BR2, D), lambda g: (0, g, 0)),
            pl.BlockSpec((_BR2, 1), lambda g: (g, 0)),
        ],
        out_shape=[
            jax.ShapeDtypeStruct((2, NPAD, D), jnp.float32),
            jax.ShapeDtypeStruct((NPAD, 1), jnp.float32),
        ],
    )(deg_parts, feat_pad, hx_pad)


def _agg_body(u_hbm, src_hbm, dst_hbm, zeros_hbm, out_hbm,
              acc, src_v, dst_v, rows, sg0, sg1, ss0, ss1):
    c = lax.axis_index("c")
    s = lax.axis_index("s")
    row0 = s * jnp.int32(RPT)
    table = u_hbm.at[c]
    pltpu.sync_copy(zeros_hbm.at[pl.ds(row0, RPT)],
                    acc.at[pl.ds(row0, RPT)])
    plsc.subcore_barrier()

    sg = (sg0, sg1)
    ss = (ss0, ss1)

    @pl.loop(0, NCH)
    def _chunk(ch):
        b0 = ch * jnp.int32(CH)
        pltpu.sync_copy(src_hbm.at[s, pl.ds(b0, CH)], src_v)
        pltpu.sync_copy(dst_hbm.at[s, pl.ds(b0, CH)], dst_v)
        gathers = {}
        scatters = {}
        for j in range(CH):
            b = j % 2
            if j >= 2:
                scatters[j - 2].wait()
            gathers[j] = pltpu.async_copy(
                table.at[src_v.at[j]], rows.at[b], sg[b])
            if j >= 1:
                pb = (j - 1) % 2
                gathers[j - 1].wait()
                scatters[j - 1] = pltpu.async_copy(
                    rows.at[pb], acc.at[dst_v.at[j - 1]], ss[pb], add=True)
        lb = (CH - 1) % 2
        gathers[CH - 1].wait()
        scatters[CH - 1] = pltpu.async_copy(
            rows.at[lb], acc.at[dst_v.at[CH - 1]], ss[lb], add=True)
        scatters[CH - 2].wait()
        scatters[CH - 1].wait()

    plsc.subcore_barrier()
    pltpu.sync_copy(acc.at[pl.ds(row0, RPT)],
                    out_hbm.at[c, pl.ds(row0, RPT)])


_BR4 = 2000


def _gru_body(agg_ref, nd_ref, hx_ref, wi_ref, bi_ref, wh_ref, bh_ref, out_ref):
    nd = nd_ref[...]
    af = agg_ref[0] * nd
    ah = agg_ref[1] * nd
    i = jnp.dot(af, wi_ref[...], preferred_element_type=jnp.float32) + bi_ref[...]
    h = jnp.dot(ah, wh_ref[...], preferred_element_type=jnp.float32) + bh_ref[...]
    i_r, i_z, i_n = jnp.split(i, 3, axis=-1)
    h_r, h_z, h_n = jnp.split(h, 3, axis=-1)
    r = jax.nn.sigmoid(i_r + h_r)
    z = jax.nn.sigmoid(i_z + h_z)
    n = jnp.tanh(i_n + r * h_n)
    out_ref[...] = (1.0 - z) * n + z * hx_ref[...]


def _gru(agg, norm_dst, hx, W_i, b_i, W_h, b_h):
    return pl.pallas_call(
        _gru_body,
        grid=(N // _BR4,),
        in_specs=[
            pl.BlockSpec((NC, _BR4, D), lambda g: (0, g, 0)),
            pl.BlockSpec((_BR4, 1), lambda g: (g, 0)),
            pl.BlockSpec((_BR4, D), lambda g: (g, 0)),
            pl.BlockSpec((D, 3 * D), lambda g: (0, 0)),
            pl.BlockSpec((1, 3 * D), lambda g: (0, 0)),
            pl.BlockSpec((D, 3 * D), lambda g: (0, 0)),
            pl.BlockSpec((1, 3 * D), lambda g: (0, 0)),
        ],
        out_specs=pl.BlockSpec((_BR4, D), lambda g: (g, 0)),
        out_shape=jax.ShapeDtypeStruct((N, D), jnp.float32),
    )(agg, norm_dst, hx, W_i, b_i, W_h, b_h)


@functools.cache
def _sc_kernels():
    mesh = plsc.VectorSubcoreMesh(
        core_axis_name="c", subcore_axis_name="s",
        num_cores=NC, num_subcores=NS)
    params = pltpu.CompilerParams(needs_layout_passes=False)
    deg_kernel = pl.kernel(
        _deg_body,
        out_type=jax.ShapeDtypeStruct((NC * NS, 2, NPAD), jnp.float32),
        mesh=mesh,
        compiler_params=params,
        scratch_types=[
            pltpu.VMEM((EPW,), jnp.int32),
            pltpu.VMEM((EPW,), jnp.int32),
            pltpu.VMEM((NPAD,), jnp.float32),
            pltpu.VMEM((NPAD,), jnp.float32),
        ],
    )
    agg_kernel = pl.kernel(
        _agg_body,
        out_type=jax.ShapeDtypeStruct((NC, NPAD, D), jnp.float32),
        mesh=mesh,
        compiler_params=params,
        scratch_types=[
            pltpu.VMEM_SHARED((NPAD, D), jnp.float32),
            pltpu.VMEM((CH, B), jnp.int32),
            pltpu.VMEM((CH, B), jnp.int32),
            pltpu.VMEM((2, B, D), jnp.float32),
            pltpu.SemaphoreType.DMA,
            pltpu.SemaphoreType.DMA,
            pltpu.SemaphoreType.DMA,
            pltpu.SemaphoreType.DMA,
        ],
    )
    return deg_kernel, agg_kernel


def kernel(feat, hx, edge_index, W_i, b_i, W_h, b_h):
    with _config.enable_x64(False):
        out = _kernel_impl(feat, hx, edge_index, W_i, b_i, W_h, b_h)
    out_dtype = jnp.result_type(jnp.promote_types(W_i.dtype, feat.dtype))
    return out.astype(out_dtype)


def _kernel_impl(feat, hx, edge_index, W_i, b_i, W_h, b_h):
    feat = feat.astype(jnp.float32)
    hx = hx.astype(jnp.float32)
    W_i = W_i.astype(jnp.float32)
    b_i = b_i.astype(jnp.float32)
    W_h = W_h.astype(jnp.float32)
    b_h = b_h.astype(jnp.float32)
    ei = edge_index.astype(jnp.int32)
    pad = EPAD - E
    src = jnp.concatenate([ei[0], jnp.full((pad,), N, jnp.int32)])
    dst = jnp.concatenate([ei[1], jnp.full((pad,), N, jnp.int32)])
    src2 = src.reshape(NC * NS, EPW)
    dst2 = dst.reshape(NC * NS, EPW)
    src3 = src.reshape(NS, NB, B)
    dst3 = dst.reshape(NS, NB, B)

    feat_pad = jnp.pad(feat, ((0, NPAD - N), (0, 0)))
    hx_pad = jnp.pad(hx, ((0, NPAD - N), (0, 0)))
    zeros = jnp.zeros((NPAD, D), jnp.float32)

    deg_kernel, agg_kernel = _sc_kernels()
    deg_parts = deg_kernel(src2, dst2)
    u, norm_dst = _scale(deg_parts, feat_pad, hx_pad)
    agg = agg_kernel(u, src3, dst3, zeros)
    return _gru(agg, norm_dst, hx, W_i, b_i.reshape(1, 3 * D),
                W_h, b_h.reshape(1, 3 * D))
